# baseline (speedup 1.0000x reference)
import functools

import numpy as np
import jax
import jax.numpy as jnp
from jax.experimental import pallas as pl
from jax.experimental.pallas import tpu as pltpu

try:
    _shard_map = jax.shard_map
except AttributeError:
    from jax.experimental.shard_map import shard_map as _shard_map

D_MODEL = 256
N_HEADS = 8
DH = D_MODEL // N_HEADS
TOP_K_N = 8
NEG_INF = float("-inf")
T_SEQ = 512
QB = 128
N_QB = T_SEQ // QB


def _attn_head_kernel(x_ref, wq_ref, wk_ref, wv_ref, wo_ref, out_ref, work_ref):
    h = pl.program_id(0)
    x = x_ref[...]
    q = jax.lax.dot_general(x, wq_ref[...], (((1,), (1,)), ((), ())),
                            preferred_element_type=jnp.float32)
    k = jax.lax.dot_general(x, wk_ref[...], (((1,), (1,)), ((), ())),
                            preferred_element_type=jnp.float32)
    v = jax.lax.dot_general(x, wv_ref[...], (((1,), (1,)), ((), ())),
                            preferred_element_type=jnp.float32)
    kt = k.T

    ctx_blocks = []
    for qb in range(N_QB):
        W = (qb + 1) * QB
        qs = q[qb * QB:(qb + 1) * QB, :]

        sc = qs[:, 0:1] + kt[0:1, :W]
        for d in range(1, DH):
            sc = jnp.maximum(sc, qs[:, d:d + 1] + kt[d:d + 1, :W])
        row = jax.lax.broadcasted_iota(jnp.int32, (QB, W), 0) + qb * QB
        col = jax.lax.broadcasted_iota(jnp.int32, (QB, W), 1)
        sc = jnp.where(col > row, NEG_INF, sc)
        work_ref[:, :W] = sc

        vals = []
        idxs = []
        for _ in range(TOP_K_N):
            a = work_ref[:, :W]
            vm = jnp.max(a, axis=1, keepdims=True)
            idx = jnp.min(jnp.where(a == vm, col, W), axis=1,
                          keepdims=True)
            work_ref[:, :W] = jnp.where(col == idx, NEG_INF, a)
            vals.append(vm)
            idxs.append(idx)

        v0 = vals[0]
        es = [jnp.exp(vm - v0) for vm in vals]
        denom = es[0]
        for e in es[1:]:
            denom = denom + e
        rden = 1.0 / denom

        num = jnp.where(col == idxs[0], es[0], 0.0)
        for m in range(1, TOP_K_N):
            num = num + jnp.where(col == idxs[m], es[m], 0.0)

        ctx_b = jnp.dot(num * rden, v[:W, :],
                        preferred_element_type=jnp.float32)
        ctx_blocks.append(ctx_b)

    ctx = jnp.concatenate(ctx_blocks, axis=0)
    contrib = jnp.dot(ctx, wo_ref[...], preferred_element_type=jnp.float32)

    @pl.when(h == 0)
    def _init():
        out_ref[...] = contrib

    @pl.when(h != 0)
    def _acc():
        out_ref[...] += contrib


def _head_pallas_call(nh_local, T, D):
    return pl.pallas_call(
        _attn_head_kernel,
        grid=(nh_local,),
        in_specs=[
            pl.BlockSpec((T, D), lambda h: (0, 0)),
            pl.BlockSpec((DH, D), lambda h: (h, 0)),
            pl.BlockSpec((DH, D), lambda h: (h, 0)),
            pl.BlockSpec((DH, D), lambda h: (h, 0)),
            pl.BlockSpec((DH, D), lambda h: (h, 0)),
        ],
        out_specs=pl.BlockSpec((T, D), lambda h: (0, 0)),
        out_shape=jax.ShapeDtypeStruct((T, D), jnp.float32),
        scratch_shapes=[
            pltpu.VMEM((QB, T), jnp.float32),
        ],
    )


@jax.jit
def kernel(x, Wq, Wk, Wv, Wo):
    B, T, D = x.shape
    x2 = x.reshape(T, D)
    devs = jax.devices()
    ndev = 2 if len(devs) >= 2 else 1
    nh_local = N_HEADS // ndev
    mesh = jax.sharding.Mesh(np.array(devs[:ndev]), ("d",))
    P = jax.sharding.PartitionSpec
    pc = _head_pallas_call(nh_local, T, D)

    def shard_fn(x2s, wqs, wks, wvs, wots):
        out = pc(x2s, wqs, wks, wvs, wots)
        return jax.lax.psum(out, "d")

    out = _shard_map(
        shard_fn,
        mesh=mesh,
        in_specs=(P(), P("d"), P("d"), P("d"), P("d")),
        out_specs=P(),
        check_vma=False,
    )(x2, Wq, Wk, Wv, Wo.T)
    return out.reshape(B, T, D)


# speedup vs baseline: 8.2576x; 8.2576x over previous
import functools

import jax
import jax.numpy as jnp
from jax.experimental import pallas as pl
from jax.experimental.pallas import tpu as pltpu

D_MODEL = 256
N_HEADS = 8
DH = D_MODEL // N_HEADS
TOP_K_N = 8
NEG_INF = float("-inf")
T_SEQ = 512
QB = 128
N_QB = T_SEQ // QB


def _make_head_kernel(nh_local):
    def _attn_head_kernel(x_ref, wq_ref, wk_ref, wv_ref, wo_ref, out_ref,
                          work_ref):
        h = pl.program_id(0)
        x = x_ref[...]
        q = jax.lax.dot_general(x, wq_ref[...], (((1,), (1,)), ((), ())),
                                preferred_element_type=jnp.float32)
        k = jax.lax.dot_general(x, wk_ref[...], (((1,), (1,)), ((), ())),
                                preferred_element_type=jnp.float32)
        v = jax.lax.dot_general(x, wv_ref[...], (((1,), (1,)), ((), ())),
                                preferred_element_type=jnp.float32)
        kt = k.T

        ctx_blocks = []
        for qb in range(N_QB):
            W = (qb + 1) * QB
            qs = q[qb * QB:(qb + 1) * QB, :]

            sc = qs[:, 0:1] + kt[0:1, :W]
            for d in range(1, DH):
                sc = jnp.maximum(sc, qs[:, d:d + 1] + kt[d:d + 1, :W])
            row = jax.lax.broadcasted_iota(jnp.int32, (QB, W), 0) + qb * QB
            col = jax.lax.broadcasted_iota(jnp.int32, (QB, W), 1)
            sc = jnp.where(col > row, NEG_INF, sc)
            work_ref[:, :W] = sc

            vals = []
            idxs = []
            for _ in range(TOP_K_N):
                a = work_ref[:, :W]
                vm = jnp.max(a, axis=1, keepdims=True)
                idx = jnp.min(jnp.where(a == vm, col, W), axis=1,
                              keepdims=True)
                work_ref[:, :W] = jnp.where(col == idx, NEG_INF, a)
                vals.append(vm)
                idxs.append(idx)

            v0 = vals[0]
            es = [jnp.exp(vm - v0) for vm in vals]
            denom = es[0]
            for e in es[1:]:
                denom = denom + e
            rden = 1.0 / denom

            num = jnp.where(col == idxs[0], es[0], 0.0)
            for m in range(1, TOP_K_N):
                num = num + jnp.where(col == idxs[m], es[m], 0.0)

            ctx_b = jnp.dot(num * rden, v[:W, :],
                            preferred_element_type=jnp.float32)
            ctx_blocks.append(ctx_b)

        ctx = jnp.concatenate(ctx_blocks, axis=0)
        contrib = jnp.dot(ctx, wo_ref[...], preferred_element_type=jnp.float32)

        @pl.when(h == 0)
        def _init():
            out_ref[...] = contrib

        @pl.when(h != 0)
        def _acc():
            out_ref[...] += contrib

    return _attn_head_kernel


def _head_pallas_call(nh_local, T, D):
    return pl.pallas_call(
        _make_head_kernel(nh_local),
        grid=(nh_local,),
        in_specs=[
            pl.BlockSpec((T, D), lambda h: (0, 0)),
            pl.BlockSpec((DH, D), lambda h: (h, 0)),
            pl.BlockSpec((DH, D), lambda h: (h, 0)),
            pl.BlockSpec((DH, D), lambda h: (h, 0)),
            pl.BlockSpec((DH, D), lambda h: (h, 0)),
        ],
        out_specs=pl.BlockSpec((T, D), lambda h: (0, 0)),
        out_shape=jax.ShapeDtypeStruct((T, D), jnp.float32),
        scratch_shapes=[
            pltpu.VMEM((QB, T), jnp.float32),
        ],
    )


@jax.jit
def kernel(x, Wq, Wk, Wv, Wo):
    B, T, D = x.shape
    x2 = x.reshape(T, D)
    out = _head_pallas_call(N_HEADS, T, D)(x2, Wq, Wk, Wv, Wo.T)
    return out.reshape(B, T, D)


# speedup vs baseline: 10.4135x; 1.2611x over previous
import functools

import jax
import jax.numpy as jnp
from jax.experimental import pallas as pl
from jax.experimental.pallas import tpu as pltpu

D_MODEL = 256
N_HEADS = 8
DH = D_MODEL // N_HEADS
TOP_K_N = 8
NEG_INF = float("-inf")
T_SEQ = 512
QB = 128
N_QB = T_SEQ // QB


def _make_head_kernel(nh_local):
    def _attn_head_kernel(x_ref, wq_ref, wk_ref, wv_ref, wo_ref, out_ref,
                          *work_refs):
        h = pl.program_id(0)
        x = x_ref[...]
        q = jax.lax.dot_general(x, wq_ref[...], (((1,), (1,)), ((), ())),
                                preferred_element_type=jnp.float32)
        k = jax.lax.dot_general(x, wk_ref[...], (((1,), (1,)), ((), ())),
                                preferred_element_type=jnp.float32)
        v = jax.lax.dot_general(x, wv_ref[...], (((1,), (1,)), ((), ())),
                                preferred_element_type=jnp.float32)
        kt = k.T

        cols = {}
        for qb in range(N_QB):
            W = (qb + 1) * QB
            qs = q[qb * QB:(qb + 1) * QB, :]

            sc = qs[:, 0:1] + kt[0:1, :W]
            for d in range(1, DH):
                sc = jnp.maximum(sc, qs[:, d:d + 1] + kt[d:d + 1, :W])
            row = jax.lax.broadcasted_iota(jnp.int32, (QB, W), 0) + qb * QB
            col = jax.lax.broadcasted_iota(jnp.int32, (QB, W), 1)
            cols[qb] = col
            work_refs[qb][...] = jnp.where(col > row, NEG_INF, sc)

        vals = {qb: [] for qb in range(N_QB)}
        idxs = {qb: [] for qb in range(N_QB)}
        for _ in range(TOP_K_N):
            for qb in range(N_QB):
                W = (qb + 1) * QB
                col = cols[qb]
                a = work_refs[qb][...]
                vm = jnp.max(a, axis=1, keepdims=True)
                idx = jnp.min(jnp.where(a == vm, col, W), axis=1,
                              keepdims=True)
                work_refs[qb][...] = jnp.where(col == idx, NEG_INF, a)
                vals[qb].append(vm)
                idxs[qb].append(idx)

        ctx_blocks = []
        for qb in range(N_QB):
            W = (qb + 1) * QB
            col = cols[qb]
            v0 = vals[qb][0]
            es = [jnp.exp(vm - v0) for vm in vals[qb]]
            denom = es[0]
            for e in es[1:]:
                denom = denom + e
            rden = 1.0 / denom

            num = jnp.where(col == idxs[qb][0], es[0], 0.0)
            for m in range(1, TOP_K_N):
                num = num + jnp.where(col == idxs[qb][m], es[m], 0.0)

            ctx_b = jnp.dot(num * rden, v[:W, :],
                            preferred_element_type=jnp.float32)
            ctx_blocks.append(ctx_b)

        ctx = jnp.concatenate(ctx_blocks, axis=0)
        contrib = jnp.dot(ctx, wo_ref[...], preferred_element_type=jnp.float32)

        @pl.when(h == 0)
        def _init():
            out_ref[...] = contrib

        @pl.when(h != 0)
        def _acc():
            out_ref[...] += contrib

    return _attn_head_kernel


def _head_pallas_call(nh_local, T, D):
    return pl.pallas_call(
        _make_head_kernel(nh_local),
        grid=(nh_local,),
        in_specs=[
            pl.BlockSpec((T, D), lambda h: (0, 0)),
            pl.BlockSpec((DH, D), lambda h: (h, 0)),
            pl.BlockSpec((DH, D), lambda h: (h, 0)),
            pl.BlockSpec((DH, D), lambda h: (h, 0)),
            pl.BlockSpec((DH, D), lambda h: (h, 0)),
        ],
        out_specs=pl.BlockSpec((T, D), lambda h: (0, 0)),
        out_shape=jax.ShapeDtypeStruct((T, D), jnp.float32),
        scratch_shapes=[
            pltpu.VMEM((QB, (qb + 1) * QB), jnp.float32)
            for qb in range(N_QB)
        ],
    )


@jax.jit
def kernel(x, Wq, Wk, Wv, Wo):
    B, T, D = x.shape
    x2 = x.reshape(T, D)
    out = _head_pallas_call(N_HEADS, T, D)(x2, Wq, Wk, Wv, Wo.T)
    return out.reshape(B, T, D)


# speedup vs baseline: 13.7722x; 1.3225x over previous
import functools

import jax
import jax.numpy as jnp
from jax.experimental import pallas as pl
from jax.experimental.pallas import tpu as pltpu

D_MODEL = 256
N_HEADS = 8
DH = D_MODEL // N_HEADS
TOP_K_N = 8
NEG_INF = float("-inf")
QB = 128


def _attn_head_kernel(x_ref, wq_ref, wk_ref, wv_ref, wo_ref, out_ref, work_ref):
    h = pl.program_id(0)
    T = x_ref.shape[0]
    n_qb = T // QB
    x = x_ref[...]
    q = jax.lax.dot_general(x, wq_ref[...], (((1,), (1,)), ((), ())),
                            preferred_element_type=jnp.float32)
    k = jax.lax.dot_general(x, wk_ref[...], (((1,), (1,)), ((), ())),
                            preferred_element_type=jnp.float32)
    v = jax.lax.dot_general(x, wv_ref[...], (((1,), (1,)), ((), ())),
                            preferred_element_type=jnp.float32)
    kt = k.T

    dr = jax.lax.broadcasted_iota(jnp.int32, (QB, QB), 0)
    dc = jax.lax.broadcasted_iota(jnp.int32, (QB, QB), 1)
    diag_mask = dc > dr

    for qb in range(n_qb):
        W = (qb + 1) * QB
        qs = q[qb * QB:(qb + 1) * QB, :]
        sc = qs[:, 0:1] + kt[0:1, :W]
        for d in range(1, DH):
            sc = jnp.maximum(sc, qs[:, d:d + 1] + kt[d:d + 1, :W])
        if qb > 0:
            work_ref[qb * QB:(qb + 1) * QB, :qb * QB] = sc[:, :qb * QB]
        work_ref[qb * QB:(qb + 1) * QB, qb * QB:W] = jnp.where(
            diag_mask, NEG_INF, sc[:, qb * QB:W])
        if W < T:
            work_ref[qb * QB:(qb + 1) * QB, W:] = jnp.full(
                (QB, T - W), NEG_INF, jnp.float32)

    colf = jax.lax.broadcasted_iota(
        jnp.int32, (T, T), 1).astype(jnp.float32)
    sent = float(T)

    vals = []
    idxs = []
    for _ in range(TOP_K_N):
        a = work_ref[...]
        vm = jnp.max(a, axis=1, keepdims=True)
        idx = jnp.min(jnp.where(a == vm, colf, sent), axis=1,
                      keepdims=True)
        work_ref[...] = jnp.where(colf == idx, NEG_INF, a)
        vals.append(vm)
        idxs.append(idx)

    v0 = vals[0]
    es = [jnp.exp(vm - v0) for vm in vals]
    denom = es[0]
    for e in es[1:]:
        denom = denom + e
    rden = 1.0 / denom

    num = jnp.where(colf == idxs[0], es[0], 0.0)
    for m in range(1, TOP_K_N):
        num = num + jnp.where(colf == idxs[m], es[m], 0.0)

    ctx = jnp.dot(num * rden, v, preferred_element_type=jnp.float32)
    contrib = jnp.dot(ctx, wo_ref[...], preferred_element_type=jnp.float32)

    @pl.when(h == 0)
    def _init():
        out_ref[...] = contrib

    @pl.when(h != 0)
    def _acc():
        out_ref[...] += contrib


@jax.jit
def kernel(x, Wq, Wk, Wv, Wo):
    B, T, D = x.shape
    x2 = x.reshape(T, D)
    out = pl.pallas_call(
        _attn_head_kernel,
        grid=(N_HEADS,),
        in_specs=[
            pl.BlockSpec((T, D), lambda h: (0, 0)),
            pl.BlockSpec((DH, D), lambda h: (h, 0)),
            pl.BlockSpec((DH, D), lambda h: (h, 0)),
            pl.BlockSpec((DH, D), lambda h: (h, 0)),
            pl.BlockSpec((DH, D), lambda h: (h, 0)),
        ],
        out_specs=pl.BlockSpec((T, D), lambda h: (0, 0)),
        out_shape=jax.ShapeDtypeStruct((T, D), jnp.float32),
        scratch_shapes=[
            pltpu.VMEM((T, T), jnp.float32),
        ],
    )(x2, Wq, Wk, Wv, Wo.T)
    return out.reshape(B, T, D)


# speedup vs baseline: 14.0283x; 1.0186x over previous
import functools

import jax
import jax.numpy as jnp
from jax.experimental import pallas as pl
from jax.experimental.pallas import tpu as pltpu

D_MODEL = 256
N_HEADS = 8
DH = D_MODEL // N_HEADS
TOP_K_N = 8
NEG_INF = float("-inf")
QB = 128


def _attn_head_kernel(x_ref, wq_ref, wk_ref, wv_ref, wo_ref, out_ref, work_ref):
    h = pl.program_id(0)
    T = x_ref.shape[0]
    n_qb = T // QB
    x = x_ref[...]
    q = jax.lax.dot_general(x, wq_ref[...], (((1,), (1,)), ((), ())),
                            preferred_element_type=jnp.float32)
    k = jax.lax.dot_general(x, wk_ref[...], (((1,), (1,)), ((), ())),
                            preferred_element_type=jnp.float32)
    v = jax.lax.dot_general(x, wv_ref[...], (((1,), (1,)), ((), ())),
                            preferred_element_type=jnp.float32)
    kt = k.T

    dr = jax.lax.broadcasted_iota(jnp.int32, (QB, QB), 0)
    dc = jax.lax.broadcasted_iota(jnp.int32, (QB, QB), 1)
    diag_mask = dc > dr

    for qb in range(n_qb):
        W = (qb + 1) * QB
        qs = q[qb * QB:(qb + 1) * QB, :]
        sc = qs[:, 0:1] + kt[0:1, :W]
        for d in range(1, DH):
            sc = jnp.maximum(sc, qs[:, d:d + 1] + kt[d:d + 1, :W])
        if qb > 0:
            work_ref[qb * QB:(qb + 1) * QB, :qb * QB] = sc[:, :qb * QB]
        work_ref[qb * QB:(qb + 1) * QB, qb * QB:W] = jnp.where(
            diag_mask, NEG_INF, sc[:, qb * QB:W])
        if W < T:
            @pl.when(h == 0)
            def _fill():
                work_ref[qb * QB:(qb + 1) * QB, W:] = jnp.full(
                    (QB, T - W), NEG_INF, jnp.float32)

    colf = jax.lax.broadcasted_iota(
        jnp.int32, (T, T), 1).astype(jnp.float32)
    sent = float(T)

    vals = []
    idxs = []
    for m in range(TOP_K_N):
        a = work_ref[...]
        vm = jnp.max(a, axis=1, keepdims=True)
        idx = jnp.min(jnp.where(a == vm, colf, sent), axis=1,
                      keepdims=True)
        if m < TOP_K_N - 1:
            work_ref[...] = jnp.where(colf == idx, NEG_INF, a)
        vals.append(vm)
        idxs.append(idx)

    v0 = vals[0]
    es = [jnp.exp(vm - v0) for vm in vals]
    denom = es[0]
    for e in es[1:]:
        denom = denom + e
    rden = 1.0 / denom

    ctx_blocks = []
    for qb in range(n_qb):
        W = (qb + 1) * QB
        r0, r1 = qb * QB, (qb + 1) * QB
        colb = colf[:QB, :W]
        num = jnp.where(colb == idxs[0][r0:r1], es[0][r0:r1], 0.0)
        for m in range(1, TOP_K_N):
            num = num + jnp.where(colb == idxs[m][r0:r1],
                                  es[m][r0:r1], 0.0)
        ctx_blocks.append(jnp.dot(num * rden[r0:r1], v[:W, :],
                                  preferred_element_type=jnp.float32))
    ctx = jnp.concatenate(ctx_blocks, axis=0)
    contrib = jnp.dot(ctx, wo_ref[...], preferred_element_type=jnp.float32)

    @pl.when(h == 0)
    def _init():
        out_ref[...] = contrib

    @pl.when(h != 0)
    def _acc():
        out_ref[...] += contrib


@jax.jit
def kernel(x, Wq, Wk, Wv, Wo):
    B, T, D = x.shape
    x2 = x.reshape(T, D)
    out = pl.pallas_call(
        _attn_head_kernel,
        grid=(N_HEADS,),
        in_specs=[
            pl.BlockSpec((T, D), lambda h: (0, 0)),
            pl.BlockSpec((DH, D), lambda h: (h, 0)),
            pl.BlockSpec((DH, D), lambda h: (h, 0)),
            pl.BlockSpec((DH, D), lambda h: (h, 0)),
            pl.BlockSpec((DH, D), lambda h: (h, 0)),
        ],
        out_specs=pl.BlockSpec((T, D), lambda h: (0, 0)),
        out_shape=jax.ShapeDtypeStruct((T, D), jnp.float32),
        scratch_shapes=[
            pltpu.VMEM((T, T), jnp.float32),
        ],
    )(x2, Wq, Wk, Wv, Wo.T)
    return out.reshape(B, T, D)


# speedup vs baseline: 14.4911x; 1.0330x over previous
import functools

import jax
import jax.numpy as jnp
from jax.experimental import pallas as pl
from jax.experimental.pallas import tpu as pltpu

D_MODEL = 256
N_HEADS = 8
DH = D_MODEL // N_HEADS
TOP_K_N = 8
NEG_INF = float("-inf")
QB = 128
HPG = 2
N_G = N_HEADS // HPG


def _attn_pair_kernel(x_ref, wq_ref, wk_ref, wv_ref, wo_ref, out_ref,
                      work0, work1):
    g = pl.program_id(0)
    T = x_ref.shape[0]
    n_qb = T // QB
    works = (work0, work1)
    x = x_ref[...]
    qq = jax.lax.dot_general(x, wq_ref[...], (((1,), (1,)), ((), ())),
                             preferred_element_type=jnp.float32)
    kk = jax.lax.dot_general(x, wk_ref[...], (((1,), (1,)), ((), ())),
                             preferred_element_type=jnp.float32)
    vv = jax.lax.dot_general(x, wv_ref[...], (((1,), (1,)), ((), ())),
                             preferred_element_type=jnp.float32)
    kt_all = kk.T

    dr = jax.lax.broadcasted_iota(jnp.int32, (QB, QB), 0)
    dc = jax.lax.broadcasted_iota(jnp.int32, (QB, QB), 1)
    diag_mask = dc > dr

    colf = jax.lax.broadcasted_iota(jnp.int32, (T, T), 1).astype(jnp.float32)
    sent = float(T)

    for hh in range(HPG):
        q = qq[:, hh * DH:(hh + 1) * DH]
        kt = kt_all[hh * DH:(hh + 1) * DH, :]
        wref = works[hh]
        for qb in range(n_qb):
            W = (qb + 1) * QB
            qs = q[qb * QB:(qb + 1) * QB, :]
            sc = qs[:, 0:1] + kt[0:1, :W]
            for d in range(1, DH):
                sc = jnp.maximum(sc, qs[:, d:d + 1] + kt[d:d + 1, :W])
            if qb > 0:
                wref[qb * QB:(qb + 1) * QB, :qb * QB] = sc[:, :qb * QB]
            wref[qb * QB:(qb + 1) * QB, qb * QB:W] = jnp.where(
                diag_mask, NEG_INF, sc[:, qb * QB:W])
            if W < T:
                @pl.when(g == 0)
                def _fill():
                    wref[qb * QB:(qb + 1) * QB, W:] = jnp.full(
                        (QB, T - W), NEG_INF, jnp.float32)

    vals = {hh: [] for hh in range(HPG)}
    idxs = {hh: [] for hh in range(HPG)}
    for m in range(TOP_K_N):
        for hh in range(HPG):
            a = works[hh][...]
            vm = jnp.max(a, axis=1, keepdims=True)
            idx = jnp.min(jnp.where(a == vm, colf, sent), axis=1,
                          keepdims=True)
            if m < TOP_K_N - 1:
                works[hh][...] = jnp.where(colf == idx, NEG_INF, a)
            vals[hh].append(vm)
            idxs[hh].append(idx)

    contrib = None
    for hh in range(HPG):
        v0 = vals[hh][0]
        es = [jnp.exp(vm - v0) for vm in vals[hh]]
        denom = es[0]
        for e in es[1:]:
            denom = denom + e
        rden = 1.0 / denom

        v = vv[:, hh * DH:(hh + 1) * DH]
        ctx_blocks = []
        for qb in range(n_qb):
            W = (qb + 1) * QB
            r0, r1 = qb * QB, (qb + 1) * QB
            colb = colf[:QB, :W]
            num = jnp.where(colb == idxs[hh][0][r0:r1],
                            es[0][r0:r1], 0.0)
            for m in range(1, TOP_K_N):
                num = num + jnp.where(colb == idxs[hh][m][r0:r1],
                                      es[m][r0:r1], 0.0)
            ctx_blocks.append(jnp.dot(num * rden[r0:r1], v[:W, :],
                                      preferred_element_type=jnp.float32))
        ctx = jnp.concatenate(ctx_blocks, axis=0)
        c = jnp.dot(ctx, wo_ref[hh * DH:(hh + 1) * DH, :],
                    preferred_element_type=jnp.float32)
        contrib = c if contrib is None else contrib + c

    @pl.when(g == 0)
    def _init():
        out_ref[...] = contrib

    @pl.when(g != 0)
    def _acc():
        out_ref[...] += contrib


@jax.jit
def kernel(x, Wq, Wk, Wv, Wo):
    B, T, D = x.shape
    x2 = x.reshape(T, D)
    out = pl.pallas_call(
        _attn_pair_kernel,
        grid=(N_G,),
        in_specs=[
            pl.BlockSpec((T, D), lambda g: (0, 0)),
            pl.BlockSpec((HPG * DH, D), lambda g: (g, 0)),
            pl.BlockSpec((HPG * DH, D), lambda g: (g, 0)),
            pl.BlockSpec((HPG * DH, D), lambda g: (g, 0)),
            pl.BlockSpec((HPG * DH, D), lambda g: (g, 0)),
        ],
        out_specs=pl.BlockSpec((T, D), lambda g: (0, 0)),
        out_shape=jax.ShapeDtypeStruct((T, D), jnp.float32),
        scratch_shapes=[
            pltpu.VMEM((T, T), jnp.float32),
            pltpu.VMEM((T, T), jnp.float32),
        ],
    )(x2, Wq, Wk, Wv, Wo.T)
    return out.reshape(B, T, D)


# speedup vs baseline: 14.8145x; 1.0223x over previous
import functools

import jax
import jax.numpy as jnp
from jax.experimental import pallas as pl
from jax.experimental.pallas import tpu as pltpu

D_MODEL = 256
N_HEADS = 8
DH = D_MODEL // N_HEADS
TOP_K_N = 8
NEG_INF = float("-inf")
QB = 128
HPG = 4
N_G = N_HEADS // HPG


def _attn_pair_kernel(x_ref, wq_ref, wk_ref, wv_ref, wo_ref, out_ref,
                      *works):
    g = pl.program_id(0)
    T = x_ref.shape[0]
    n_qb = T // QB
    x = x_ref[...]
    qq = jax.lax.dot_general(x, wq_ref[...], (((1,), (1,)), ((), ())),
                             preferred_element_type=jnp.float32)
    kk = jax.lax.dot_general(x, wk_ref[...], (((1,), (1,)), ((), ())),
                             preferred_element_type=jnp.float32)
    vv = jax.lax.dot_general(x, wv_ref[...], (((1,), (1,)), ((), ())),
                             preferred_element_type=jnp.float32)
    kt_all = kk.T

    dr = jax.lax.broadcasted_iota(jnp.int32, (QB, QB), 0)
    dc = jax.lax.broadcasted_iota(jnp.int32, (QB, QB), 1)
    diag_mask = dc > dr

    colf = jax.lax.broadcasted_iota(jnp.int32, (T, T), 1).astype(jnp.float32)
    sent = float(T)

    for hh in range(HPG):
        q = qq[:, hh * DH:(hh + 1) * DH]
        kt = kt_all[hh * DH:(hh + 1) * DH, :]
        wref = works[hh]
        for qb in range(n_qb):
            W = (qb + 1) * QB
            qs = q[qb * QB:(qb + 1) * QB, :]
            sc = qs[:, 0:1] + kt[0:1, :W]
            for d in range(1, DH):
                sc = jnp.maximum(sc, qs[:, d:d + 1] + kt[d:d + 1, :W])
            if qb > 0:
                wref[qb * QB:(qb + 1) * QB, :qb * QB] = sc[:, :qb * QB]
            wref[qb * QB:(qb + 1) * QB, qb * QB:W] = jnp.where(
                diag_mask, NEG_INF, sc[:, qb * QB:W])
            if W < T:
                @pl.when(g == 0)
                def _fill():
                    wref[qb * QB:(qb + 1) * QB, W:] = jnp.full(
                        (QB, T - W), NEG_INF, jnp.float32)

    vals = {hh: [] for hh in range(HPG)}
    idxs = {hh: [] for hh in range(HPG)}
    for m in range(TOP_K_N):
        for hh in range(HPG):
            a = works[hh][...]
            vm = jnp.max(a, axis=1, keepdims=True)
            idx = jnp.min(jnp.where(a == vm, colf, sent), axis=1,
                          keepdims=True)
            if m < TOP_K_N - 1:
                works[hh][...] = jnp.where(colf == idx, NEG_INF, a)
            vals[hh].append(vm)
            idxs[hh].append(idx)

    contrib = None
    for hh in range(HPG):
        v0 = vals[hh][0]
        es = [jnp.exp(vm - v0) for vm in vals[hh]]
        denom = es[0]
        for e in es[1:]:
            denom = denom + e
        rden = 1.0 / denom

        v = vv[:, hh * DH:(hh + 1) * DH]
        ctx_blocks = []
        for qb in range(n_qb):
            W = (qb + 1) * QB
            r0, r1 = qb * QB, (qb + 1) * QB
            colb = colf[:QB, :W]
            num = jnp.where(colb == idxs[hh][0][r0:r1],
                            es[0][r0:r1], 0.0)
            for m in range(1, TOP_K_N):
                num = num + jnp.where(colb == idxs[hh][m][r0:r1],
                                      es[m][r0:r1], 0.0)
            ctx_blocks.append(jnp.dot(num * rden[r0:r1], v[:W, :],
                                      preferred_element_type=jnp.float32))
        ctx = jnp.concatenate(ctx_blocks, axis=0)
        c = jnp.dot(ctx, wo_ref[hh * DH:(hh + 1) * DH, :],
                    preferred_element_type=jnp.float32)
        contrib = c if contrib is None else contrib + c

    @pl.when(g == 0)
    def _init():
        out_ref[...] = contrib

    @pl.when(g != 0)
    def _acc():
        out_ref[...] += contrib


@jax.jit
def kernel(x, Wq, Wk, Wv, Wo):
    B, T, D = x.shape
    x2 = x.reshape(T, D)
    out = pl.pallas_call(
        _attn_pair_kernel,
        grid=(N_G,),
        in_specs=[
            pl.BlockSpec((T, D), lambda g: (0, 0)),
            pl.BlockSpec((HPG * DH, D), lambda g: (g, 0)),
            pl.BlockSpec((HPG * DH, D), lambda g: (g, 0)),
            pl.BlockSpec((HPG * DH, D), lambda g: (g, 0)),
            pl.BlockSpec((HPG * DH, D), lambda g: (g, 0)),
        ],
        out_specs=pl.BlockSpec((T, D), lambda g: (0, 0)),
        out_shape=jax.ShapeDtypeStruct((T, D), jnp.float32),
        scratch_shapes=[
            pltpu.VMEM((T, T), jnp.float32) for _ in range(HPG)
        ],
    )(x2, Wq, Wk, Wv, Wo.T)
    return out.reshape(B, T, D)


# speedup vs baseline: 14.9581x; 1.0097x over previous
import functools

import jax
import jax.numpy as jnp
from jax.experimental import pallas as pl
from jax.experimental.pallas import tpu as pltpu

D_MODEL = 256
N_HEADS = 8
DH = D_MODEL // N_HEADS
TOP_K_N = 8
NEG_INF = float("-inf")
QB = 128
HPG = 8
N_G = N_HEADS // HPG


def _attn_pair_kernel(x_ref, wq_ref, wk_ref, wv_ref, wo_ref, out_ref,
                      *works):
    g = pl.program_id(0)
    T = x_ref.shape[0]
    n_qb = T // QB
    x = x_ref[...]
    qq = jax.lax.dot_general(x, wq_ref[...], (((1,), (1,)), ((), ())),
                             preferred_element_type=jnp.float32)
    kk = jax.lax.dot_general(x, wk_ref[...], (((1,), (1,)), ((), ())),
                             preferred_element_type=jnp.float32)
    vv = jax.lax.dot_general(x, wv_ref[...], (((1,), (1,)), ((), ())),
                             preferred_element_type=jnp.float32)
    kt_all = kk.T

    dr = jax.lax.broadcasted_iota(jnp.int32, (QB, QB), 0)
    dc = jax.lax.broadcasted_iota(jnp.int32, (QB, QB), 1)
    diag_mask = dc > dr

    colf = jax.lax.broadcasted_iota(jnp.int32, (T, T), 1).astype(jnp.float32)
    sent = float(T)

    for hh in range(HPG):
        q = qq[:, hh * DH:(hh + 1) * DH]
        kt = kt_all[hh * DH:(hh + 1) * DH, :]
        wref = works[hh]
        for qb in range(n_qb):
            W = (qb + 1) * QB
            qs = q[qb * QB:(qb + 1) * QB, :]
            sc = qs[:, 0:1] + kt[0:1, :W]
            for d in range(1, DH):
                sc = jnp.maximum(sc, qs[:, d:d + 1] + kt[d:d + 1, :W])
            if qb > 0:
                wref[qb * QB:(qb + 1) * QB, :qb * QB] = sc[:, :qb * QB]
            wref[qb * QB:(qb + 1) * QB, qb * QB:W] = jnp.where(
                diag_mask, NEG_INF, sc[:, qb * QB:W])
            if W < T:
                @pl.when(g == 0)
                def _fill():
                    wref[qb * QB:(qb + 1) * QB, W:] = jnp.full(
                        (QB, T - W), NEG_INF, jnp.float32)

    vals = {hh: [] for hh in range(HPG)}
    idxs = {hh: [] for hh in range(HPG)}
    for m in range(TOP_K_N):
        for hh in range(HPG):
            a = works[hh][...]
            vm = jnp.max(a, axis=1, keepdims=True)
            idx = jnp.min(jnp.where(a == vm, colf, sent), axis=1,
                          keepdims=True)
            if m < TOP_K_N - 1:
                works[hh][...] = jnp.where(colf == idx, NEG_INF, a)
            vals[hh].append(vm)
            idxs[hh].append(idx)

    contrib = None
    for hh in range(HPG):
        v0 = vals[hh][0]
        es = [jnp.exp(vm - v0) for vm in vals[hh]]
        denom = es[0]
        for e in es[1:]:
            denom = denom + e
        rden = 1.0 / denom

        v = vv[:, hh * DH:(hh + 1) * DH]
        ctx_blocks = []
        for qb in range(n_qb):
            W = (qb + 1) * QB
            r0, r1 = qb * QB, (qb + 1) * QB
            colb = colf[:QB, :W]
            num = jnp.where(colb == idxs[hh][0][r0:r1],
                            es[0][r0:r1], 0.0)
            for m in range(1, TOP_K_N):
                num = num + jnp.where(colb == idxs[hh][m][r0:r1],
                                      es[m][r0:r1], 0.0)
            ctx_blocks.append(jnp.dot(num * rden[r0:r1], v[:W, :],
                                      preferred_element_type=jnp.float32))
        ctx = jnp.concatenate(ctx_blocks, axis=0)
        c = jnp.dot(ctx, wo_ref[hh * DH:(hh + 1) * DH, :],
                    preferred_element_type=jnp.float32)
        contrib = c if contrib is None else contrib + c

    @pl.when(g == 0)
    def _init():
        out_ref[...] = contrib

    @pl.when(g != 0)
    def _acc():
        out_ref[...] += contrib


@jax.jit
def kernel(x, Wq, Wk, Wv, Wo):
    B, T, D = x.shape
    x2 = x.reshape(T, D)
    out = pl.pallas_call(
        _attn_pair_kernel,
        grid=(N_G,),
        in_specs=[
            pl.BlockSpec((T, D), lambda g: (0, 0)),
            pl.BlockSpec((HPG * DH, D), lambda g: (g, 0)),
            pl.BlockSpec((HPG * DH, D), lambda g: (g, 0)),
            pl.BlockSpec((HPG * DH, D), lambda g: (g, 0)),
            pl.BlockSpec((HPG * DH, D), lambda g: (g, 0)),
        ],
        out_specs=pl.BlockSpec((T, D), lambda g: (0, 0)),
        out_shape=jax.ShapeDtypeStruct((T, D), jnp.float32),
        scratch_shapes=[
            pltpu.VMEM((T, T), jnp.float32) for _ in range(HPG)
        ],
    )(x2, Wq, Wk, Wv, Wo.T)
    return out.reshape(B, T, D)
